# Initial kernel scaffold; baseline (speedup 1.0000x reference)
import jax
import jax.numpy as jnp
from jax.experimental import pallas as pl


def kernel(ids, table):
    raise NotImplementedError("write your pallas kernel here")


# speedup vs baseline: 1.8393x; 1.8393x over previous
import functools

import jax
import jax.numpy as jnp
from jax import lax
from jax.experimental import pallas as pl
from jax.experimental.pallas import tpu as pltpu
from jax.experimental.pallas import tpu_sc as plsc

NUM_ENTITIES = 1000000
D = 64
BATCH = 16384
HIST = 50

B = BATCH * HIST
NC, NS = 2, 16
NW = NC * NS
PER_W = B // NW
CHUNK = 128
N_CHUNK = PER_W // CHUNK

_mesh = plsc.VectorSubcoreMesh(core_axis_name="c", subcore_axis_name="s")


@functools.partial(
    pl.kernel,
    mesh=_mesh,
    compiler_params=pltpu.CompilerParams(use_tc_tiling_on_sc=False),
    out_type=jax.ShapeDtypeStruct((B, D), jnp.float32),
    scratch_types=[
        pltpu.VMEM((N_CHUNK, CHUNK), jnp.int32),
        pltpu.VMEM((CHUNK, D), jnp.float32),
        pltpu.VMEM((CHUNK, D), jnp.float32),
        pltpu.SemaphoreType.DMA,
        pltpu.SemaphoreType.DMA,
    ],
)
def _gather_sc(ids_hbm, table_hbm, out_hbm, idx_v, rows0, rows1, sem0, sem1):
    wid = lax.axis_index("s") * NC + lax.axis_index("c")
    base = wid * PER_W

    pltpu.sync_copy(ids_hbm.at[wid], idx_v)

    def gather(g, buf, sem):
        pltpu.async_copy(table_hbm.at[idx_v.at[g]], buf, sem)

    def gather_wait(g, buf, sem):
        pltpu.make_async_copy(table_hbm.at[idx_v.at[g]], buf, sem).wait()

    def write(g, buf):
        pltpu.sync_copy(buf, out_hbm.at[pl.ds(base + g * CHUNK, CHUNK)])

    gather(0, rows0, sem0)

    def body(i, carry):
        g = i * 2
        gather(g + 1, rows1, sem1)
        gather_wait(g, rows0, sem0)
        write(g, rows0)

        @pl.when(g + 2 < N_CHUNK)
        def _():
            gather(g + 2, rows0, sem0)

        gather_wait(g + 1, rows1, sem1)
        write(g + 1, rows1)
        return carry

    lax.fori_loop(0, N_CHUNK // 2, body, 0)


def kernel(ids, table):
    ids_r = ids.reshape(NW, N_CHUNK, CHUNK)
    out = _gather_sc(ids_r, table)
    return out.reshape(BATCH, HIST, D)


# speedup vs baseline: 1.8765x; 1.0202x over previous
import functools

import jax
import jax.numpy as jnp
from jax import lax
from jax.experimental import pallas as pl
from jax.experimental.pallas import tpu as pltpu
from jax.experimental.pallas import tpu_sc as plsc

NUM_ENTITIES = 1000000
D = 64
BATCH = 16384
HIST = 50

B = BATCH * HIST
NC, NS = 2, 16
NW = NC * NS
PER_W = B // NW
CHUNK = 128
N_CHUNK = PER_W // CHUNK
NBUF = 8
AHEAD = 4

_mesh = plsc.VectorSubcoreMesh(core_axis_name="c", subcore_axis_name="s")


@functools.partial(
    pl.kernel,
    mesh=_mesh,
    compiler_params=pltpu.CompilerParams(use_tc_tiling_on_sc=False),
    out_type=jax.ShapeDtypeStruct((B, D), jnp.float32),
    scratch_types=[
        pltpu.VMEM((N_CHUNK, CHUNK), jnp.int32),
        [pltpu.VMEM((CHUNK, D), jnp.float32) for _ in range(NBUF)],
        [pltpu.SemaphoreType.DMA for _ in range(NBUF)],
        [pltpu.SemaphoreType.DMA for _ in range(NBUF)],
    ],
)
def _gather_sc(ids_hbm, table_hbm, out_hbm, idx_v, bufs, gsems, wsems):
    wid = lax.axis_index("s") * NC + lax.axis_index("c")
    base = wid * PER_W

    pltpu.sync_copy(ids_hbm.at[wid], idx_v)

    def gather(g, slot):
        pltpu.async_copy(table_hbm.at[idx_v.at[g]], bufs[slot], gsems[slot])

    def gather_wait(g, slot):
        pltpu.make_async_copy(
            table_hbm.at[idx_v.at[g]], bufs[slot], gsems[slot]).wait()

    def write(g, slot):
        pltpu.async_copy(
            bufs[slot], out_hbm.at[pl.ds(base + g * CHUNK, CHUNK)], wsems[slot])

    def write_wait(g, slot):
        pltpu.make_async_copy(
            bufs[slot], out_hbm.at[pl.ds(base + g * CHUNK, CHUNK)],
            wsems[slot]).wait()

    for g0 in range(AHEAD):
        gather(g0, g0)

    def body(i, carry):
        gbase = i * NBUF
        for b in range(NBUF):
            g = gbase + b
            slot_p = (b + AHEAD) % NBUF
            gp = g + AHEAD

            @pl.when(gp < N_CHUNK)
            def _(gp=gp, slot_p=slot_p):
                @pl.when(gp >= NBUF)
                def _():
                    write_wait(gp - NBUF, slot_p)
                gather(gp, slot_p)

            gather_wait(g, b)
            write(g, b)
        return carry

    lax.fori_loop(0, N_CHUNK // NBUF, body, 0)

    for b in range(NBUF):
        write_wait(0, b)


def kernel(ids, table):
    ids_r = ids.reshape(NW, N_CHUNK, CHUNK)
    out = _gather_sc(ids_r, table)
    return out.reshape(BATCH, HIST, D)
